# baseline (speedup 1.0000x reference)
import jax
import jax.numpy as jnp
from jax import lax
from jax.experimental import pallas as pl
from jax.experimental.pallas import tpu as pltpu
from jax.experimental.pallas import tpu_sc as plsc

BATCH = 64
SEQ = 4096
L = 16
NCHUNK = SEQ // L
MAX_MASKED = 615
KCHUNKS = (MAX_MASKED + L - 1) // L
NWORKERS = 32
ROWS_PER_W = BATCH // NWORKERS

S_SIZE = 0
S_RANK = 1
S_PREF = 2
S_SHIFT = 3
S_DONE = 4
S_KEFF = 5


def _iota16():
  return lax.iota(jnp.int32, 16)


def _scalar(x_vec):
  return lax.squeeze(lax.slice(x_vec, (0,), (1,)), (0,))


def _row_compute(o, ids_v, rand_v, key_v, cbuf_a, cbuf_b, hist, suf, out_v, st):
  lanes = _iota16()
  ones = jnp.ones((L,), jnp.int32)

  def p0_body(c, cnt):
    ids = ids_v[pl.ds(o + c * L, L)]
    rb = lax.bitcast_convert_type(rand_v[pl.ds(o + c * L, L)], jnp.int32)
    m = ids != 0
    ukey = jnp.where(m, rb + 1, 0)
    key_v[pl.ds(c * L, L)] = ukey
    bucket = jnp.right_shift(ukey, 24)
    plsc.addupdate_scatter(hist, [bucket * L + lanes], ones)
    return cnt + jnp.where(m, 1, 0)

  cnt = plsc.parallel_loop(0, NCHUNK, unroll=8,
                           carry=jnp.zeros((L,), jnp.int32))(p0_body)
  n_tok = jnp.sum(cnt)

  t_f = n_tok.astype(jnp.float32) * jnp.float32(0.15)
  t_i = t_f.astype(jnp.int32)
  target = t_i + jnp.where(t_i.astype(jnp.float32) < t_f, 1, 0)

  st[S_KEFF] = MAX_MASKED

  @pl.when(n_tok != SEQ)
  def _keff_slow():
    def keff_body(c, carry):
      run, kacc = carry
      m_i = jnp.where(key_v[pl.ds(c * L, L)] > 0, 1, 0)
      cum = plsc.cumsum(m_i) + run
      pos = lanes + c * L
      keep = jnp.where((cum <= target) & (pos < MAX_MASKED), 1, 0)
      run = run + jnp.sum(m_i)
      return run, kacc + keep

    _, kacc = plsc.parallel_loop(
        0, KCHUNKS, carry=(jnp.int32(0), jnp.zeros((L,), jnp.int32)))(
            keff_body)
    st[S_KEFF] = jnp.sum(kacc)

  st[S_SIZE] = SEQ
  st[S_RANK] = st[S_KEFF]
  st[S_PREF] = 0
  st[S_SHIFT] = 24
  st[S_DONE] = 0

  def suffix_and_search(rank):
    zeros = jnp.zeros((L,), jnp.int32)

    def suf_body(i, cum):
      b = 255 - i
      cum = cum + hist[pl.ds(b * L, L)]
      suf[pl.ds(b * L, L)] = cum
      hist[pl.ds(b * L, L)] = zeros
      return cum

    plsc.parallel_loop(0, 256, unroll=8, carry=zeros)(suf_body)

    def bs_body(i, lo):
      step = jnp.int32(128) >> i
      mid = lo + step
      s_mid = jnp.where(
          mid < 256, jnp.sum(suf[pl.ds(jnp.minimum(mid, 255) * L, L)]), 0)
      return jnp.where(s_mid >= rank, mid, lo)

    bstar = lax.fori_loop(0, 8, bs_body, jnp.int32(0))
    s_b = jnp.sum(suf[pl.ds(bstar * L, L)])
    c_above = jnp.where(
        bstar < 255, jnp.sum(suf[pl.ds(jnp.minimum(bstar + 1, 255) * L, L)]), 0)
    return bstar, c_above, s_b - c_above

  def compact(src, dst, n, shift, bsel, with_hist):
    static_n = isinstance(n, int)
    nch = (n + L - 1) // L
    nshift = jnp.broadcast_to(shift - 8, (L,))
    shift_b = jnp.broadcast_to(shift, (L,))

    def body(c, off):
      v = src[pl.ds(c * L, L)]
      byte = jnp.bitwise_and(jnp.right_shift(v, shift_b), 255)
      m = byte == bsel
      if not static_n:
        m = m & ((lanes + c * L) < n)
      plsc.store_compressed(dst.at[pl.ds(off, L)], v, mask=m)
      if with_hist:
        nbyte = jnp.bitwise_and(jnp.right_shift(v, nshift), 255)
        plsc.addupdate_scatter(hist, [nbyte * L + lanes], ones, mask=m)
      npick = plsc.all_reduce_population_count(m)
      return off + _scalar(npick)

    return plsc.parallel_loop(0, nch, unroll=8, carry=jnp.int32(0))(body)

  def level(src, dst, is_last, first=False):
    n = SEQ if first else st[S_SIZE]
    rank = st[S_RANK]
    shift = st[S_SHIFT]

    bstar, c_above, tot = suffix_and_search(rank)
    new_rank = rank - c_above
    new_pref = st[S_PREF] * 256 + bstar
    new_shift = shift - 8

    done_now = (new_rank == tot) | (new_shift < 0)
    st[S_RANK] = new_rank
    st[S_PREF] = new_pref
    st[S_SHIFT] = jnp.where(done_now, shift, new_shift)

    @pl.when(jnp.logical_not(done_now) & (tot <= L))
    def _sort_exit():
      compact(src, dst, n, shift, bstar, with_hist=False)
      vals = dst[pl.ds(0, L)]
      mvalid = lanes < tot
      sk, _, _ = plsc.sort_key_val(vals, vals, mask=mvalid, descending=True)
      dst[pl.ds(0, L)] = sk
      t_val = _scalar(dst[pl.ds(new_rank - 1, L)])
      gt_cnt = jnp.sum(jnp.where(mvalid & (vals > t_val), 1, 0))
      st[S_RANK] = new_rank - gt_cnt
      st[S_PREF] = t_val
      st[S_SHIFT] = 0
      st[S_DONE] = 1

    if not is_last:
      @pl.when(jnp.logical_not(done_now) & (tot > L))
      def _compact_next():
        st[S_SIZE] = compact(src, dst, n, shift, bstar, with_hist=True)

    @pl.when(done_now)
    def _done():
      st[S_DONE] = 1

  plan = [(key_v, cbuf_a), (cbuf_a, cbuf_b), (cbuf_b, cbuf_a),
          (cbuf_a, cbuf_b)]
  for li, (src, dst) in enumerate(plan):
    if li == 0:
      level(src, dst, is_last=False, first=True)
    else:
      @pl.when(st[S_DONE] == 0)
      def _run_level(src=src, dst=dst, last=(li == 3)):
        level(src, dst, is_last=last)

  q_shift = jnp.broadcast_to(st[S_SHIFT], (L,))
  q_pref = st[S_PREF]
  r_fin = st[S_RANK]

  def fin_body(c, run_eq):
    ukey = key_v[pl.ds(c * L, L)]
    q = jnp.right_shift(ukey, q_shift)
    gt = q > q_pref
    eq = q == q_pref
    eq_i = jnp.where(eq, 1, 0)
    rank_vec = plsc.cumsum(eq_i) + run_eq
    sel = gt | (eq & (rank_vec <= r_fin))
    out_v[pl.ds(o + c * L, L)] = jnp.where(sel, 1, 0)
    return run_eq + plsc.all_reduce_population_count(eq)

  plsc.parallel_loop(0, NCHUNK, unroll=8,
                     carry=jnp.zeros((L,), jnp.int32))(fin_body)


def _sc_kernel(ids_hbm, rand_hbm, out_hbm,
               ids_v, rand_v, key_v, cbuf_a, cbuf_b,
               hist, suf, out_v, st, sem_a, sem_b, sem_o):
  wid = lax.axis_index("s") * 2 + lax.axis_index("c")
  base0 = (wid * ROWS_PER_W) * SEQ

  cp_i0 = pltpu.async_copy(
      ids_hbm.at[pl.ds(base0, SEQ)], ids_v.at[pl.ds(0, SEQ)], sem_a)
  cp_r0 = pltpu.async_copy(
      rand_hbm.at[pl.ds(base0, SEQ)], rand_v.at[pl.ds(0, SEQ)], sem_a)
  cp_i1 = pltpu.async_copy(
      ids_hbm.at[pl.ds(base0 + SEQ, SEQ)], ids_v.at[pl.ds(SEQ, SEQ)], sem_b)
  cp_r1 = pltpu.async_copy(
      rand_hbm.at[pl.ds(base0 + SEQ, SEQ)], rand_v.at[pl.ds(SEQ, SEQ)], sem_b)

  def z_body(b):
    hist[pl.ds(b * L, L)] = jnp.zeros((L,), jnp.int32)

  plsc.parallel_loop(0, 256, unroll=4)(z_body)

  cp_i0.wait()
  cp_r0.wait()

  def row_body(i, _):
    @pl.when(i == 1)
    def _wait_row1():
      cp_i1.wait()
      cp_r1.wait()

    o = i * SEQ
    _row_compute(o, ids_v, rand_v, key_v, cbuf_a, cbuf_b, hist, suf, out_v,
                 st)
    pltpu.async_copy(out_v.at[pl.ds(o, SEQ)],
                     out_hbm.at[pl.ds(base0 + o, SEQ)], sem_o)
    return 0

  lax.fori_loop(0, ROWS_PER_W, row_body, 0)

  pltpu.make_async_copy(out_v.at[pl.ds(0, SEQ)],
                        out_hbm.at[pl.ds(base0, SEQ)], sem_o).wait()
  pltpu.make_async_copy(out_v.at[pl.ds(SEQ, SEQ)],
                        out_hbm.at[pl.ds(base0 + SEQ, SEQ)], sem_o).wait()


@jax.jit
def kernel(input_ids, rand):
  mesh = plsc.VectorSubcoreMesh(
      core_axis_name="c", subcore_axis_name="s", num_cores=2, num_subcores=16)
  run = pl.kernel(
      _sc_kernel,
      out_type=jax.ShapeDtypeStruct((BATCH * SEQ,), jnp.int32),
      mesh=mesh,
      compiler_params=pltpu.CompilerParams(needs_layout_passes=False),
      scratch_types=[
          pltpu.VMEM((2 * SEQ,), jnp.int32),
          pltpu.VMEM((2 * SEQ,), jnp.float32),
          pltpu.VMEM((SEQ,), jnp.int32),
          pltpu.VMEM((SEQ + L,), jnp.int32),
          pltpu.VMEM((SEQ + L,), jnp.int32),
          pltpu.VMEM((256 * L,), jnp.int32),
          pltpu.VMEM((256 * L,), jnp.int32),
          pltpu.VMEM((2 * SEQ,), jnp.int32),
          pltpu.SMEM((8,), jnp.int32),
          pltpu.SemaphoreType.DMA,
          pltpu.SemaphoreType.DMA,
          pltpu.SemaphoreType.DMA,
      ],
  )
  ids_flat = input_ids.reshape(BATCH * SEQ).astype(jnp.int32)
  rand_flat = rand.reshape(BATCH * SEQ)
  out = run(ids_flat, rand_flat)
  return out.reshape(BATCH, SEQ).astype(jnp.bool_)


# speedup vs baseline: 1.0296x; 1.0296x over previous
import jax
import jax.numpy as jnp
from jax import lax
from jax.experimental import pallas as pl
from jax.experimental.pallas import tpu as pltpu
from jax.experimental.pallas import tpu_sc as plsc

BATCH = 64
SEQ = 4096
L = 16
NCHUNK = SEQ // L
MAX_MASKED = 615
KCHUNKS = (MAX_MASKED + L - 1) // L
NWORKERS = 32
ROWS_PER_W = BATCH // NWORKERS

S_SIZE = 0
S_RANK = 1
S_PREF = 2
S_SHIFT = 3
S_DONE = 4
S_KEFF = 5


def _iota16():
  return lax.iota(jnp.int32, 16)


def _scalar(x_vec):
  return lax.squeeze(lax.slice(x_vec, (0,), (1,)), (0,))


def _row_compute(o, ids_v, rand_v, key_v, cbuf_a, cbuf_b, hist, suf, out_v, st):
  lanes = _iota16()
  ones = jnp.ones((L,), jnp.int32)

  def p0_body(c, cnt):
    ids = ids_v[pl.ds(o + c * L, L)]
    rb = lax.bitcast_convert_type(rand_v[pl.ds(o + c * L, L)], jnp.int32)
    m = ids != 0
    ukey = jnp.where(m, rb + 1, 0)
    key_v[pl.ds(c * L, L)] = ukey
    bucket = jnp.right_shift(ukey, 24)
    plsc.addupdate_scatter(hist, [bucket * L + lanes], ones)
    return cnt + jnp.where(m, 1, 0)

  cnt = plsc.parallel_loop(0, NCHUNK, unroll=4,
                           carry=jnp.zeros((L,), jnp.int32))(p0_body)
  n_tok = jnp.sum(cnt)

  t_f = n_tok.astype(jnp.float32) * jnp.float32(0.15)
  t_i = t_f.astype(jnp.int32)
  target = t_i + jnp.where(t_i.astype(jnp.float32) < t_f, 1, 0)

  st[S_KEFF] = MAX_MASKED

  @pl.when(n_tok != SEQ)
  def _keff_slow():
    def keff_body(c, carry):
      run, kacc = carry
      m_i = jnp.where(key_v[pl.ds(c * L, L)] > 0, 1, 0)
      cum = plsc.cumsum(m_i) + run
      pos = lanes + c * L
      keep = jnp.where((cum <= target) & (pos < MAX_MASKED), 1, 0)
      run = run + jnp.sum(m_i)
      return run, kacc + keep

    _, kacc = plsc.parallel_loop(
        0, KCHUNKS, carry=(jnp.int32(0), jnp.zeros((L,), jnp.int32)))(
            keff_body)
    st[S_KEFF] = jnp.sum(kacc)

  st[S_SIZE] = SEQ
  st[S_RANK] = st[S_KEFF]
  st[S_PREF] = 0
  st[S_SHIFT] = 24
  st[S_DONE] = 0

  def suffix_and_search(rank):
    zeros = jnp.zeros((L,), jnp.int32)

    def suf_body(i, cum):
      b = 255 - i
      cum = cum + hist[pl.ds(b * L, L)]
      suf[pl.ds(b * L, L)] = cum
      hist[pl.ds(b * L, L)] = zeros
      return cum

    plsc.parallel_loop(0, 256, unroll=4, carry=zeros)(suf_body)

    def bs_body(i, lo):
      step = jnp.int32(128) >> i
      mid = lo + step
      s_mid = jnp.where(
          mid < 256, jnp.sum(suf[pl.ds(jnp.minimum(mid, 255) * L, L)]), 0)
      return jnp.where(s_mid >= rank, mid, lo)

    bstar = lax.fori_loop(0, 8, bs_body, jnp.int32(0))
    s_b = jnp.sum(suf[pl.ds(bstar * L, L)])
    c_above = jnp.where(
        bstar < 255, jnp.sum(suf[pl.ds(jnp.minimum(bstar + 1, 255) * L, L)]), 0)
    return bstar, c_above, s_b - c_above

  def compact(src, dst, n, shift, bsel, with_hist):
    static_n = isinstance(n, int)
    nch = (n + L - 1) // L
    nshift = jnp.broadcast_to(shift - 8, (L,))
    shift_b = jnp.broadcast_to(shift, (L,))

    def body(c, off):
      v = src[pl.ds(c * L, L)]
      byte = jnp.bitwise_and(jnp.right_shift(v, shift_b), 255)
      m = byte == bsel
      if not static_n:
        m = m & ((lanes + c * L) < n)
      plsc.store_compressed(dst.at[pl.ds(off, L)], v, mask=m)
      if with_hist:
        nbyte = jnp.bitwise_and(jnp.right_shift(v, nshift), 255)
        plsc.addupdate_scatter(hist, [nbyte * L + lanes], ones, mask=m)
      npick = plsc.all_reduce_population_count(m)
      return off + _scalar(npick)

    return plsc.parallel_loop(0, nch, unroll=4, carry=jnp.int32(0))(body)

  def level(src, dst, is_last, first=False):
    n = SEQ if first else st[S_SIZE]
    rank = st[S_RANK]
    shift = st[S_SHIFT]

    bstar, c_above, tot = suffix_and_search(rank)
    new_rank = rank - c_above
    new_pref = st[S_PREF] * 256 + bstar
    new_shift = shift - 8

    done_now = (new_rank == tot) | (new_shift < 0)
    st[S_RANK] = new_rank
    st[S_PREF] = new_pref
    st[S_SHIFT] = jnp.where(done_now, shift, new_shift)

    @pl.when(jnp.logical_not(done_now) & (tot <= L))
    def _sort_exit():
      compact(src, dst, n, shift, bstar, with_hist=False)
      vals = dst[pl.ds(0, L)]
      mvalid = lanes < tot
      sk, _, _ = plsc.sort_key_val(vals, vals, mask=mvalid, descending=True)
      dst[pl.ds(0, L)] = sk
      t_val = _scalar(dst[pl.ds(new_rank - 1, L)])
      gt_cnt = jnp.sum(jnp.where(mvalid & (vals > t_val), 1, 0))
      st[S_RANK] = new_rank - gt_cnt
      st[S_PREF] = t_val
      st[S_SHIFT] = 0
      st[S_DONE] = 1

    if not is_last:
      @pl.when(jnp.logical_not(done_now) & (tot > L))
      def _compact_next():
        st[S_SIZE] = compact(src, dst, n, shift, bstar, with_hist=True)

    @pl.when(done_now)
    def _done():
      st[S_DONE] = 1

  plan = [(key_v, cbuf_a), (cbuf_a, cbuf_b), (cbuf_b, cbuf_a),
          (cbuf_a, cbuf_b)]
  for li, (src, dst) in enumerate(plan):
    if li == 0:
      level(src, dst, is_last=False, first=True)
    else:
      @pl.when(st[S_DONE] == 0)
      def _run_level(src=src, dst=dst, last=(li == 3)):
        level(src, dst, is_last=last)

  q_shift = jnp.broadcast_to(st[S_SHIFT], (L,))
  q_pref = st[S_PREF]
  r_fin = st[S_RANK]

  def fin_body(c, run_eq):
    ukey = key_v[pl.ds(c * L, L)]
    q = jnp.right_shift(ukey, q_shift)
    gt = q > q_pref
    eq = q == q_pref
    eq_i = jnp.where(eq, 1, 0)
    rank_vec = plsc.cumsum(eq_i) + run_eq
    sel = gt | (eq & (rank_vec <= r_fin))
    out_v[pl.ds(o + c * L, L)] = jnp.where(sel, 1, 0)
    return run_eq + plsc.all_reduce_population_count(eq)

  plsc.parallel_loop(0, NCHUNK, unroll=4,
                     carry=jnp.zeros((L,), jnp.int32))(fin_body)


def _sc_kernel(ids_hbm, rand_hbm, out_hbm,
               ids_v, rand_v, key_v, cbuf_a, cbuf_b,
               hist, suf, out_v, st, sem_a, sem_b, sem_o):
  wid = lax.axis_index("s") * 2 + lax.axis_index("c")
  base0 = (wid * ROWS_PER_W) * SEQ

  cp_i0 = pltpu.async_copy(
      ids_hbm.at[pl.ds(base0, SEQ)], ids_v.at[pl.ds(0, SEQ)], sem_a)
  cp_r0 = pltpu.async_copy(
      rand_hbm.at[pl.ds(base0, SEQ)], rand_v.at[pl.ds(0, SEQ)], sem_a)
  cp_i1 = pltpu.async_copy(
      ids_hbm.at[pl.ds(base0 + SEQ, SEQ)], ids_v.at[pl.ds(SEQ, SEQ)], sem_b)
  cp_r1 = pltpu.async_copy(
      rand_hbm.at[pl.ds(base0 + SEQ, SEQ)], rand_v.at[pl.ds(SEQ, SEQ)], sem_b)

  def z_body(b):
    hist[pl.ds(b * L, L)] = jnp.zeros((L,), jnp.int32)

  plsc.parallel_loop(0, 256, unroll=4)(z_body)

  cp_i0.wait()
  cp_r0.wait()

  def row_body(i, _):
    @pl.when(i == 1)
    def _wait_row1():
      cp_i1.wait()
      cp_r1.wait()

    o = i * SEQ
    _row_compute(o, ids_v, rand_v, key_v, cbuf_a, cbuf_b, hist, suf, out_v,
                 st)
    pltpu.async_copy(out_v.at[pl.ds(o, SEQ)],
                     out_hbm.at[pl.ds(base0 + o, SEQ)], sem_o)
    return 0

  lax.fori_loop(0, ROWS_PER_W, row_body, 0)

  pltpu.make_async_copy(out_v.at[pl.ds(0, SEQ)],
                        out_hbm.at[pl.ds(base0, SEQ)], sem_o).wait()
  pltpu.make_async_copy(out_v.at[pl.ds(SEQ, SEQ)],
                        out_hbm.at[pl.ds(base0 + SEQ, SEQ)], sem_o).wait()


@jax.jit
def kernel(input_ids, rand):
  mesh = plsc.VectorSubcoreMesh(
      core_axis_name="c", subcore_axis_name="s", num_cores=2, num_subcores=16)
  run = pl.kernel(
      _sc_kernel,
      out_type=jax.ShapeDtypeStruct((BATCH * SEQ,), jnp.int32),
      mesh=mesh,
      compiler_params=pltpu.CompilerParams(needs_layout_passes=False),
      scratch_types=[
          pltpu.VMEM((2 * SEQ,), jnp.int32),
          pltpu.VMEM((2 * SEQ,), jnp.float32),
          pltpu.VMEM((SEQ,), jnp.int32),
          pltpu.VMEM((SEQ + L,), jnp.int32),
          pltpu.VMEM((SEQ + L,), jnp.int32),
          pltpu.VMEM((256 * L,), jnp.int32),
          pltpu.VMEM((256 * L,), jnp.int32),
          pltpu.VMEM((2 * SEQ,), jnp.int32),
          pltpu.SMEM((8,), jnp.int32),
          pltpu.SemaphoreType.DMA,
          pltpu.SemaphoreType.DMA,
          pltpu.SemaphoreType.DMA,
      ],
  )
  ids_flat = input_ids.reshape(BATCH * SEQ).astype(jnp.int32)
  rand_flat = rand.reshape(BATCH * SEQ)
  out = run(ids_flat, rand_flat)
  return out.reshape(BATCH, SEQ).astype(jnp.bool_)
